# Initial kernel scaffold; baseline (speedup 1.0000x reference)
import jax
import jax.numpy as jnp
from jax.experimental import pallas as pl


def kernel(x, edge_index, Wself0, Wneigh0, b0, Wself1, Wneigh1, b1, Wself2, Wneigh2, b2):
    raise NotImplementedError("write your pallas kernel here")


# speedup vs baseline: 6.5051x; 6.5051x over previous
import functools

import jax
import jax.numpy as jnp
from jax import lax
from jax.experimental import pallas as pl
from jax.experimental.pallas import tpu as pltpu
from jax.experimental.pallas import tpu_sc as plsc

N = 10000
E = 320000
D = 128
CLASSES = 40

N_PAD = 10112
N_SC = 2
N_TILES = 16
NW = N_SC * N_TILES
CHUNK = 128
EPW = 10112
E_PAD = NW * EPW
CHUNKS_PER_W = EPW // CHUNK
RPT = N_PAD // N_TILES

W0 = 144
W1 = 128
W2 = 48

_MESH = plsc.VectorSubcoreMesh(core_axis_name="c", subcore_axis_name="s")


def _make_agg(width):
  @functools.partial(
      pl.kernel,
      out_type=jax.ShapeDtypeStruct((N_SC * N_PAD, width), jnp.float32),
      mesh=_MESH,
      compiler_params=pltpu.CompilerParams(use_tc_tiling_on_sc=False),
      scratch_types=[
          pltpu.VMEM((CHUNK,), jnp.int32),
          pltpu.VMEM((CHUNK,), jnp.int32),
          pltpu.VMEM((CHUNK, width), jnp.float32),
          pltpu.VMEM_SHARED((N_PAD, width), jnp.float32),
          pltpu.SemaphoreType.DMA,
      ],
  )
  def agg(t_hbm, src_hbm, dst_hbm, zeros_hbm, out_hbm,
          src_v, dst_v, rows_v, acc_sh, sem):
    c = lax.axis_index("c")
    s = lax.axis_index("s")
    wid = c * N_TILES + s
    r0 = s * RPT
    pltpu.sync_copy(zeros_hbm.at[pl.ds(r0, RPT)], acc_sh.at[pl.ds(r0, RPT)])
    plsc.subcore_barrier()

    def body(i, carry):
      base = wid * EPW + i * CHUNK
      pltpu.sync_copy(src_hbm.at[pl.ds(base, CHUNK)], src_v)
      pltpu.sync_copy(dst_hbm.at[pl.ds(base, CHUNK)], dst_v)
      pltpu.async_copy(t_hbm.at[src_v], rows_v, sem).wait()
      pltpu.sync_copy(rows_v, acc_sh.at[dst_v], add=True)
      return carry

    lax.fori_loop(0, CHUNKS_PER_W, body, 0)
    plsc.subcore_barrier()
    pltpu.sync_copy(acc_sh.at[pl.ds(r0, RPT)],
                    out_hbm.at[pl.ds(c * N_PAD + r0, RPT)])

  return agg


_agg0 = _make_agg(W0)
_agg1 = _make_agg(W1)
_agg2 = _make_agg(W2)


_R = 1264
_G = N_PAD // _R


def _mm0(x_pad, wn0_pad):
  def body(x_ref, w_ref, o_ref):
    mm = jnp.dot(x_ref[...], w_ref[...], preferred_element_type=jnp.float32)
    col = lax.broadcasted_iota(jnp.int32, (1, W0), 1)
    o_ref[...] = mm + (col == D).astype(jnp.float32)

  return pl.pallas_call(
      body,
      grid=(_G,),
      in_specs=[pl.BlockSpec((_R, D), lambda i: (i, 0)),
                pl.BlockSpec((D, W0), lambda i: (0, 0))],
      out_specs=pl.BlockSpec((_R, W0), lambda i: (i, 0)),
      out_shape=jax.ShapeDtypeStruct((N_PAD, W0), jnp.float32),
  )(x_pad, wn0_pad)


def _combine0(x_pad, s0, wself0, b0, wneigh1):
  def body(x_ref, sa_ref, sb_ref, ws_ref, b_ref, wn_ref,
           h1_ref, t1_ref, rdeg_ref):
    sm = sa_ref[0] + sb_ref[0]
    rdeg = 1.0 / jnp.maximum(sm[:, D:D + 1], 1.0)
    neigh = sm[:, :D] * rdeg
    h1 = jnp.maximum(
        jnp.dot(x_ref[...], ws_ref[...], preferred_element_type=jnp.float32)
        + neigh + b_ref[...], 0.0)
    h1_ref[...] = h1
    t1_ref[...] = jnp.dot(h1, wn_ref[...], preferred_element_type=jnp.float32)
    rdeg_ref[...] = rdeg

  s3 = s0.reshape(N_SC, N_PAD, W0)
  return pl.pallas_call(
      body,
      grid=(_G,),
      in_specs=[
          pl.BlockSpec((_R, D), lambda i: (i, 0)),
          pl.BlockSpec((1, _R, W0), lambda i: (0, i, 0)),
          pl.BlockSpec((1, _R, W0), lambda i: (1, i, 0)),
          pl.BlockSpec((D, D), lambda i: (0, 0)),
          pl.BlockSpec((1, D), lambda i: (0, 0)),
          pl.BlockSpec((D, D), lambda i: (0, 0)),
      ],
      out_specs=[
          pl.BlockSpec((_R, D), lambda i: (i, 0)),
          pl.BlockSpec((_R, D), lambda i: (i, 0)),
          pl.BlockSpec((_R, 1), lambda i: (i, 0)),
      ],
      out_shape=[
          jax.ShapeDtypeStruct((N_PAD, D), jnp.float32),
          jax.ShapeDtypeStruct((N_PAD, D), jnp.float32),
          jax.ShapeDtypeStruct((N_PAD, 1), jnp.float32),
      ],
  )(x_pad, s3, s3, wself0, b0, wneigh1)


def _combine1(h1, s1, rdeg, wself1, b1, wneigh2_pad, wself2):
  def body(h_ref, sa_ref, sb_ref, rd_ref, ws_ref, b_ref, wn_ref, w2_ref,
           t2_ref, u2_ref):
    neigh = (sa_ref[0] + sb_ref[0]) * rd_ref[...]
    h2 = jnp.maximum(
        jnp.dot(h_ref[...], ws_ref[...], preferred_element_type=jnp.float32)
        + neigh + b_ref[...], 0.0)
    t2_ref[...] = jnp.dot(h2, wn_ref[...], preferred_element_type=jnp.float32)
    u2_ref[...] = jnp.dot(h2, w2_ref[...], preferred_element_type=jnp.float32)

  s3 = s1.reshape(N_SC, N_PAD, W1)
  return pl.pallas_call(
      body,
      grid=(_G,),
      in_specs=[
          pl.BlockSpec((_R, D), lambda i: (i, 0)),
          pl.BlockSpec((1, _R, W1), lambda i: (0, i, 0)),
          pl.BlockSpec((1, _R, W1), lambda i: (1, i, 0)),
          pl.BlockSpec((_R, 1), lambda i: (i, 0)),
          pl.BlockSpec((D, D), lambda i: (0, 0)),
          pl.BlockSpec((1, D), lambda i: (0, 0)),
          pl.BlockSpec((D, W2), lambda i: (0, 0)),
          pl.BlockSpec((D, CLASSES), lambda i: (0, 0)),
      ],
      out_specs=[
          pl.BlockSpec((_R, W2), lambda i: (i, 0)),
          pl.BlockSpec((_R, CLASSES), lambda i: (i, 0)),
      ],
      out_shape=[
          jax.ShapeDtypeStruct((N_PAD, W2), jnp.float32),
          jax.ShapeDtypeStruct((N_PAD, CLASSES), jnp.float32),
      ],
  )(h1, s3, s3, rdeg, wself1, b1, wneigh2_pad, wself2)


def _combine2(u2, s2, rdeg, b2):
  R2 = 1000
  def body(u_ref, sa_ref, sb_ref, rd_ref, b_ref, o_ref):
    sm = (sa_ref[0] + sb_ref[0])[:, :CLASSES]
    o_ref[...] = u_ref[...] + sm * rd_ref[...] + b_ref[...]

  s3 = s2.reshape(N_SC, N_PAD, W2)
  return pl.pallas_call(
      body,
      grid=(N // R2,),
      in_specs=[
          pl.BlockSpec((R2, CLASSES), lambda i: (i, 0)),
          pl.BlockSpec((1, R2, W2), lambda i: (0, i, 0)),
          pl.BlockSpec((1, R2, W2), lambda i: (1, i, 0)),
          pl.BlockSpec((R2, 1), lambda i: (i, 0)),
          pl.BlockSpec((1, CLASSES), lambda i: (0, 0)),
      ],
      out_specs=pl.BlockSpec((R2, CLASSES), lambda i: (i, 0)),
      out_shape=jax.ShapeDtypeStruct((N, CLASSES), jnp.float32),
  )(u2, s3, s3, rdeg, b2)


def kernel(x, edge_index, Wself0, Wneigh0, b0, Wself1, Wneigh1, b1,
           Wself2, Wneigh2, b2):
  x_pad = jnp.pad(x, ((0, N_PAD - N), (0, 0)))
  src = edge_index[0]
  dst = edge_index[1]
  npad_e = E_PAD - E
  pad_ids = jnp.arange(npad_e, dtype=jnp.int32)
  src_pad = jnp.concatenate([src, (pad_ids * 97) % N])
  dst_pad = jnp.concatenate([dst, N + pad_ids % (N_PAD - N)])

  wn0_pad = jnp.pad(Wneigh0, ((0, 0), (0, W0 - D)))
  wn2_pad = jnp.pad(Wneigh2, ((0, 0), (0, W2 - CLASSES)))
  z0 = jnp.zeros((N_PAD, W0), jnp.float32)
  z1 = jnp.zeros((N_PAD, W1), jnp.float32)
  z2 = jnp.zeros((N_PAD, W2), jnp.float32)

  t0 = _mm0(x_pad, wn0_pad)
  s0 = _agg0(t0, src_pad, dst_pad, z0)
  h1, t1, rdeg = _combine0(x_pad, s0, Wself0, b0.reshape(1, D), Wneigh1)
  s1 = _agg1(t1, src_pad, dst_pad, z1)
  t2, u2 = _combine1(h1, s1, rdeg, Wself1, b1.reshape(1, D), wn2_pad, Wself2)
  s2 = _agg2(t2, src_pad, dst_pad, z2)
  return _combine2(u2, s2, rdeg, b2.reshape(1, CLASSES))


# speedup vs baseline: 13.1555x; 2.0223x over previous
import functools

import jax
import jax.numpy as jnp
from jax import lax
from jax.experimental import pallas as pl
from jax.experimental.pallas import tpu as pltpu
from jax.experimental.pallas import tpu_sc as plsc

N = 10000
E = 320000
D = 128
CLASSES = 40

N_PAD = 10112
N_SC = 2
N_TILES = 16
NW = N_SC * N_TILES
CHUNK = 128
CHUNKS_PER_W = 80
EPW = CHUNKS_PER_W * CHUNK
E_PAD = NW * EPW
RPT = N_PAD // N_TILES

W2 = 48
WD = 16

_MESH = plsc.VectorSubcoreMesh(core_axis_name="c", subcore_axis_name="s")


def _make_agg(width, tc_tiling):
  @functools.partial(
      pl.kernel,
      out_type=jax.ShapeDtypeStruct((N_SC * N_PAD, width), jnp.float32),
      mesh=_MESH,
      compiler_params=pltpu.CompilerParams(use_tc_tiling_on_sc=tc_tiling),
      scratch_types=[
          pltpu.VMEM((EPW,), jnp.int32),
          pltpu.VMEM((CHUNK,), jnp.int32),
          pltpu.VMEM((CHUNK,), jnp.int32),
          pltpu.VMEM((CHUNK, width), jnp.float32),
          pltpu.VMEM((CHUNK, width), jnp.float32),
          pltpu.VMEM_SHARED((N_PAD, width), jnp.float32),
          pltpu.SemaphoreType.DMA,
          pltpu.SemaphoreType.DMA,
      ],
  )
  def agg(t_hbm, src_hbm, dst_hbm, zeros_hbm, out_hbm,
          src_v, dst0_v, dst1_v, rows0_v, rows1_v, acc_sh, gsem, dsem):
    c = lax.axis_index("c")
    s = lax.axis_index("s")
    wid = c * N_TILES + s
    r0 = s * RPT
    e0 = wid * EPW
    pltpu.sync_copy(zeros_hbm.at[pl.ds(r0, RPT)], acc_sh.at[pl.ds(r0, RPT)])
    pltpu.sync_copy(src_hbm.at[pl.ds(e0, EPW)], src_v)
    plsc.subcore_barrier()

    def gather(i, buf):
      return pltpu.async_copy(
          t_hbm.at[src_v.at[pl.ds(i * CHUNK, CHUNK)]], buf, gsem)

    def dstload(i, buf):
      return pltpu.async_copy(
          dst_hbm.at[pl.ds(e0 + i * CHUNK, CHUNK)], buf, dsem)

    dstload(0, dst0_v)
    gather(0, rows0_v)

    def body(g, carry):
      i0 = 2 * g
      dstload(i0 + 1, dst1_v)
      gather(i0 + 1, rows1_v)
      pltpu.make_async_copy(t_hbm.at[src_v.at[pl.ds(0, CHUNK)]],
                            rows0_v, gsem).wait()
      pltpu.make_async_copy(dst_hbm.at[pl.ds(e0, CHUNK)], dst0_v, dsem).wait()
      pltpu.sync_copy(rows0_v, acc_sh.at[dst0_v], add=True)

      @pl.when(i0 + 2 < CHUNKS_PER_W)
      def _():
        dstload(i0 + 2, dst0_v)
        gather(i0 + 2, rows0_v)

      pltpu.make_async_copy(t_hbm.at[src_v.at[pl.ds(0, CHUNK)]],
                            rows1_v, gsem).wait()
      pltpu.make_async_copy(dst_hbm.at[pl.ds(e0, CHUNK)], dst1_v, dsem).wait()
      pltpu.sync_copy(rows1_v, acc_sh.at[dst1_v], add=True)
      return carry

    lax.fori_loop(0, CHUNKS_PER_W // 2, body, 0)
    plsc.subcore_barrier()
    pltpu.sync_copy(acc_sh.at[pl.ds(r0, RPT)],
                    out_hbm.at[pl.ds(c * N_PAD + r0, RPT)])

  return agg


@functools.partial(
    pl.kernel,
    out_type=jax.ShapeDtypeStruct((N_SC * N_PAD, WD), jnp.float32),
    mesh=_MESH,
    compiler_params=pltpu.CompilerParams(use_tc_tiling_on_sc=False),
    scratch_types=[
        pltpu.VMEM((CHUNK, WD), jnp.float32),
        pltpu.VMEM((CHUNK,), jnp.int32),
        pltpu.VMEM((CHUNK,), jnp.int32),
        pltpu.VMEM_SHARED((N_PAD, WD), jnp.float32),
        pltpu.SemaphoreType.DMA,
    ],
)
def _deg(ones_hbm, dst_hbm, zeros_hbm, out_hbm,
         ones_v, dst0_v, dst1_v, acc_sh, dsem):
  c = lax.axis_index("c")
  s = lax.axis_index("s")
  wid = c * N_TILES + s
  r0 = s * RPT
  e0 = wid * EPW
  pltpu.sync_copy(zeros_hbm.at[pl.ds(r0, RPT)], acc_sh.at[pl.ds(r0, RPT)])
  pltpu.sync_copy(ones_hbm, ones_v)
  plsc.subcore_barrier()

  def dstload(i, buf):
    return pltpu.async_copy(
        dst_hbm.at[pl.ds(e0 + i * CHUNK, CHUNK)], buf, dsem)

  dstload(0, dst0_v)

  def body(g, carry):
    i0 = 2 * g
    dstload(i0 + 1, dst1_v)
    pltpu.make_async_copy(dst_hbm.at[pl.ds(e0, CHUNK)], dst0_v, dsem).wait()
    pltpu.sync_copy(ones_v, acc_sh.at[dst0_v], add=True)

    @pl.when(i0 + 2 < CHUNKS_PER_W)
    def _():
      dstload(i0 + 2, dst0_v)

    pltpu.make_async_copy(dst_hbm.at[pl.ds(e0, CHUNK)], dst1_v, dsem).wait()
    pltpu.sync_copy(ones_v, acc_sh.at[dst1_v], add=True)
    return carry

  lax.fori_loop(0, CHUNKS_PER_W // 2, body, 0)
  plsc.subcore_barrier()
  pltpu.sync_copy(acc_sh.at[pl.ds(r0, RPT)],
                  out_hbm.at[pl.ds(c * N_PAD + r0, RPT)])


_agg0 = _make_agg(D, True)
_agg2 = _make_agg(W2, False)


_R = 1264
_G = N_PAD // _R


def _mm0(x_pad, wn0):
  def body(x_ref, w_ref, o_ref):
    o_ref[...] = jnp.dot(x_ref[...], w_ref[...],
                         preferred_element_type=jnp.float32)

  return pl.pallas_call(
      body,
      grid=(_G,),
      in_specs=[pl.BlockSpec((_R, D), lambda i: (i, 0)),
                pl.BlockSpec((D, D), lambda i: (0, 0))],
      out_specs=pl.BlockSpec((_R, D), lambda i: (i, 0)),
      out_shape=jax.ShapeDtypeStruct((N_PAD, D), jnp.float32),
  )(x_pad, wn0)


def _combine0(x_pad, s0, sdeg, wself0, b0, wneigh1):
  def body(x_ref, sa_ref, sb_ref, da_ref, db_ref, ws_ref, b_ref, wn_ref,
           h1_ref, t1_ref, rdeg_ref):
    deg = da_ref[0][:, 0:1] + db_ref[0][:, 0:1]
    rdeg = 1.0 / jnp.maximum(deg, 1.0)
    neigh = (sa_ref[0] + sb_ref[0]) * rdeg
    h1 = jnp.maximum(
        jnp.dot(x_ref[...], ws_ref[...], preferred_element_type=jnp.float32)
        + neigh + b_ref[...], 0.0)
    h1_ref[...] = h1
    t1_ref[...] = jnp.dot(h1, wn_ref[...], preferred_element_type=jnp.float32)
    rdeg_ref[...] = rdeg

  s3 = s0.reshape(N_SC, N_PAD, D)
  d3 = sdeg.reshape(N_SC, N_PAD, WD)
  return pl.pallas_call(
      body,
      grid=(_G,),
      in_specs=[
          pl.BlockSpec((_R, D), lambda i: (i, 0)),
          pl.BlockSpec((1, _R, D), lambda i: (0, i, 0)),
          pl.BlockSpec((1, _R, D), lambda i: (1, i, 0)),
          pl.BlockSpec((1, _R, WD), lambda i: (0, i, 0)),
          pl.BlockSpec((1, _R, WD), lambda i: (1, i, 0)),
          pl.BlockSpec((D, D), lambda i: (0, 0)),
          pl.BlockSpec((1, D), lambda i: (0, 0)),
          pl.BlockSpec((D, D), lambda i: (0, 0)),
      ],
      out_specs=[
          pl.BlockSpec((_R, D), lambda i: (i, 0)),
          pl.BlockSpec((_R, D), lambda i: (i, 0)),
          pl.BlockSpec((_R, 1), lambda i: (i, 0)),
      ],
      out_shape=[
          jax.ShapeDtypeStruct((N_PAD, D), jnp.float32),
          jax.ShapeDtypeStruct((N_PAD, D), jnp.float32),
          jax.ShapeDtypeStruct((N_PAD, 1), jnp.float32),
      ],
  )(x_pad, s3, s3, d3, d3, wself0, b0, wneigh1)


def _combine1(h1, s1, rdeg, wself1, b1, wneigh2_pad, wself2):
  def body(h_ref, sa_ref, sb_ref, rd_ref, ws_ref, b_ref, wn_ref, w2_ref,
           t2_ref, u2_ref):
    neigh = (sa_ref[0] + sb_ref[0]) * rd_ref[...]
    h2 = jnp.maximum(
        jnp.dot(h_ref[...], ws_ref[...], preferred_element_type=jnp.float32)
        + neigh + b_ref[...], 0.0)
    t2_ref[...] = jnp.dot(h2, wn_ref[...], preferred_element_type=jnp.float32)
    u2_ref[...] = jnp.dot(h2, w2_ref[...], preferred_element_type=jnp.float32)

  s3 = s1.reshape(N_SC, N_PAD, D)
  return pl.pallas_call(
      body,
      grid=(_G,),
      in_specs=[
          pl.BlockSpec((_R, D), lambda i: (i, 0)),
          pl.BlockSpec((1, _R, D), lambda i: (0, i, 0)),
          pl.BlockSpec((1, _R, D), lambda i: (1, i, 0)),
          pl.BlockSpec((_R, 1), lambda i: (i, 0)),
          pl.BlockSpec((D, D), lambda i: (0, 0)),
          pl.BlockSpec((1, D), lambda i: (0, 0)),
          pl.BlockSpec((D, W2), lambda i: (0, 0)),
          pl.BlockSpec((D, CLASSES), lambda i: (0, 0)),
      ],
      out_specs=[
          pl.BlockSpec((_R, W2), lambda i: (i, 0)),
          pl.BlockSpec((_R, CLASSES), lambda i: (i, 0)),
      ],
      out_shape=[
          jax.ShapeDtypeStruct((N_PAD, W2), jnp.float32),
          jax.ShapeDtypeStruct((N_PAD, CLASSES), jnp.float32),
      ],
  )(h1, s3, s3, rdeg, wself1, b1, wneigh2_pad, wself2)


def _combine2(u2, s2, rdeg, b2):
  R2 = 1000
  def body(u_ref, sa_ref, sb_ref, rd_ref, b_ref, o_ref):
    sm = (sa_ref[0] + sb_ref[0])[:, :CLASSES]
    o_ref[...] = u_ref[...] + sm * rd_ref[...] + b_ref[...]

  s3 = s2.reshape(N_SC, N_PAD, W2)
  return pl.pallas_call(
      body,
      grid=(N // R2,),
      in_specs=[
          pl.BlockSpec((R2, CLASSES), lambda i: (i, 0)),
          pl.BlockSpec((1, R2, W2), lambda i: (0, i, 0)),
          pl.BlockSpec((1, R2, W2), lambda i: (1, i, 0)),
          pl.BlockSpec((R2, 1), lambda i: (i, 0)),
          pl.BlockSpec((1, CLASSES), lambda i: (0, 0)),
      ],
      out_specs=pl.BlockSpec((R2, CLASSES), lambda i: (i, 0)),
      out_shape=jax.ShapeDtypeStruct((N, CLASSES), jnp.float32),
  )(u2, s3, s3, rdeg, b2)


def kernel(x, edge_index, Wself0, Wneigh0, b0, Wself1, Wneigh1, b1,
           Wself2, Wneigh2, b2):
  x_pad = jnp.pad(x, ((0, N_PAD - N), (0, 0)))
  src = edge_index[0]
  dst = edge_index[1]
  npad_e = E_PAD - E
  pad_ids = jnp.arange(npad_e, dtype=jnp.int32)
  src_pad = jnp.concatenate([src, (pad_ids * 97) % N])
  dst_pad = jnp.concatenate([dst, N + pad_ids % (N_PAD - N)])

  wn2_pad = jnp.pad(Wneigh2, ((0, 0), (0, W2 - CLASSES)))
  z128 = jnp.zeros((N_PAD, D), jnp.float32)
  z48 = jnp.zeros((N_PAD, W2), jnp.float32)
  z16 = jnp.zeros((N_PAD, WD), jnp.float32)
  ones16 = jnp.ones((CHUNK, WD), jnp.float32)

  sdeg = _deg(ones16, dst_pad, z16)
  t0 = _mm0(x_pad, Wneigh0)
  s0 = _agg0(t0, src_pad, dst_pad, z128)
  h1, t1, rdeg = _combine0(x_pad, s0, sdeg, Wself0, b0.reshape(1, D), Wneigh1)
  s1 = _agg0(t1, src_pad, dst_pad, z128)
  t2, u2 = _combine1(h1, s1, rdeg, Wself1, b1.reshape(1, D), wn2_pad, Wself2)
  s2 = _agg2(t2, src_pad, dst_pad, z48)
  return _combine2(u2, s2, rdeg, b2.reshape(1, CLASSES))
